# Initial kernel scaffold; baseline (speedup 1.0000x reference)
import jax
import jax.numpy as jnp
from jax.experimental import pallas as pl


def kernel(x, edge_index, edge_weight, z, emb_table, emb_gn_scale, emb_gn_bias, trans_W, trans_b, comb_W, comb_b, conv_gn_scale, conv_gn_bias, gns_scale, gns_bias):
    raise NotImplementedError("write your pallas kernel here")


# speedup vs baseline: 2.9821x; 2.9821x over previous
import functools

import jax
import jax.numpy as jnp
from jax import lax
from jax.experimental import pallas as pl
from jax.experimental.pallas import tpu as pltpu
from jax.experimental.pallas import tpu_sc as plsc

N = 10000
E = 160000
H = 256
HH = 128
ZR = 0.8

NC = 2
NS = 16
EPT = E // (NS)
CHUNK = 80
NCHUNK = EPT // CHUNK

BLK = 1000
GRID = N // BLK

@functools.cache
def _mesh():
    return plsc.VectorSubcoreMesh(core_axis_name="c", subcore_axis_name="s",
                                  num_cores=NC, num_subcores=NS)


def _split_bf16(x):
    bits = lax.bitcast_convert_type(x, jnp.uint32)
    hi = lax.bitcast_convert_type(bits & jnp.uint32(0xFFFF0000), jnp.float32)
    lo = (x - hi) * 256.0
    return hi.astype(jnp.bfloat16), lo.astype(jnp.bfloat16)


def _mm(h, w):
    h_hi, h_lo = _split_bf16(h)
    w_hi, w_lo = _split_bf16(w)
    out = jnp.dot(h_hi, w_hi, preferred_element_type=jnp.float32)
    out += jnp.dot(h_lo, w_hi, preferred_element_type=jnp.float32) * (1.0 / 256.0)
    out += jnp.dot(h_hi, w_lo, preferred_element_type=jnp.float32) * (1.0 / 256.0)
    return out


def _blend(mask, a0, a1):
    return (1.0 - ZR) * (a0 + a1) + (2.0 * ZR - 1.0) * jnp.where(mask, a1, a0)


def _affine_from_stats(stats_ref, scale_ref, bias_ref):
    mean = stats_ref[0:1, :] * (1.0 / N)
    ex2 = stats_ref[1:2, :] * (1.0 / N)
    var = ex2 - mean * mean
    a = scale_ref[...] / jnp.sqrt(var + 1e-5)
    c = bias_ref[...] - mean * a
    return a, c


def _emb_body(x_ref, emb_ref, h_ref, stats_ref):
    i = pl.program_id(0)
    lanes = lax.broadcasted_iota(jnp.int32, (1, 128), 1)
    oh = (x_ref[...] == lanes).astype(jnp.float32)
    h = _mm(oh, emb_ref[...])

    @pl.when(i == 0)
    def _():
        stats_ref[...] = jnp.zeros_like(stats_ref)

    h_ref[...] = h
    stats_ref[0:1, :] += jnp.sum(h, axis=0, keepdims=True)
    stats_ref[1:2, :] += jnp.sum(h * h, axis=0, keepdims=True)


def _tc_emb(x2d, emb_pad):
    return pl.pallas_call(
        _emb_body,
        grid=(GRID,),
        in_specs=[
            pl.BlockSpec((BLK, 1), lambda i: (i, 0)),
            pl.BlockSpec((128, H), lambda i: (0, 0)),
        ],
        out_specs=[
            pl.BlockSpec((BLK, H), lambda i: (i, 0)),
            pl.BlockSpec((8, H), lambda i: (0, 0)),
        ],
        out_shape=[
            jax.ShapeDtypeStruct((N, H), jnp.float32),
            jax.ShapeDtypeStruct((8, H), jnp.float32),
        ],
        compiler_params=pltpu.CompilerParams(
            dimension_semantics=("arbitrary",)),
    )(x2d, emb_pad)


def _trans_body(u_ref, stats_ref, gs_ref, gb_ref, z_ref, wt_ref, bt_ref,
                h_ref, hm_ref, *, apply_relu):
    a, c = _affine_from_stats(stats_ref, gs_ref, gb_ref)
    h = u_ref[...] * a + c
    if apply_relu:
        h = jnp.maximum(h, 0.0)
    h_ref[...] = h
    xx = _mm(h, wt_ref[...]) + bt_ref[...]
    xx = jnp.maximum(xx, 0.0)
    mask = z_ref[...] > 0.5
    hm_ref[...] = _blend(mask, xx[:, :H], xx[:, H:])


def _tc_trans(u, stats, gs, gb, z2d, wt_bf16, bt, apply_relu):
    return pl.pallas_call(
        functools.partial(_trans_body, apply_relu=apply_relu),
        grid=(GRID,),
        in_specs=[
            pl.BlockSpec((BLK, H), lambda i: (i, 0)),
            pl.BlockSpec((8, H), lambda i: (0, 0)),
            pl.BlockSpec((1, H), lambda i: (0, 0)),
            pl.BlockSpec((1, H), lambda i: (0, 0)),
            pl.BlockSpec((BLK, 1), lambda i: (i, 0)),
            pl.BlockSpec((H, 2 * H), lambda i: (0, 0)),
            pl.BlockSpec((1, 2 * H), lambda i: (0, 0)),
        ],
        out_specs=[
            pl.BlockSpec((BLK, H), lambda i: (i, 0)),
            pl.BlockSpec((BLK, H), lambda i: (i, 0)),
        ],
        out_shape=[
            jax.ShapeDtypeStruct((N, H), jnp.float32),
            jax.ShapeDtypeStruct((N, H), jnp.float32),
        ],
    )(u, stats, gs, gb, z2d, wt_bf16, bt)


def _inv_deg(deg_ref):
    d = deg_ref[..., 0:1] + deg_ref[..., 1:2]
    d = jnp.where(d < 0.5, d + 1.0, d)
    return 1.0 / d


def _p1_body(agg_lo_ref, agg_hi_ref, deg_ref, st_lo_ref, st_hi_ref):
    i = pl.program_id(0)
    inv = _inv_deg(deg_ref)

    @pl.when(i == 0)
    def _():
        st_lo_ref[...] = jnp.zeros_like(st_lo_ref)
        st_hi_ref[...] = jnp.zeros_like(st_hi_ref)

    for aref, sref in ((agg_lo_ref, st_lo_ref), (agg_hi_ref, st_hi_ref)):
        s = aref[0] * inv
        sref[0:1, :] += jnp.sum(s, axis=0, keepdims=True)
        sref[1:2, :] += jnp.sum(s * s, axis=0, keepdims=True)


def _tc_p1(agg, deg2):
    return pl.pallas_call(
        _p1_body,
        grid=(GRID,),
        in_specs=[
            pl.BlockSpec((1, BLK, HH), lambda i: (0, i, 0)),
            pl.BlockSpec((1, BLK, HH), lambda i: (1, i, 0)),
            pl.BlockSpec((BLK, 2), lambda i: (i, 0)),
        ],
        out_specs=[
            pl.BlockSpec((8, HH), lambda i: (0, 0)),
            pl.BlockSpec((8, HH), lambda i: (0, 0)),
        ],
        out_shape=[
            jax.ShapeDtypeStruct((8, HH), jnp.float32),
            jax.ShapeDtypeStruct((8, HH), jnp.float32),
        ],
        compiler_params=pltpu.CompilerParams(
            dimension_semantics=("arbitrary",)),
    )(agg, agg, deg2)


def _p2_body(agg_lo_ref, agg_hi_ref, deg_ref, st_lo_ref, st_hi_ref,
             cgs_lo_ref, cgb_lo_ref, cgs_hi_ref, cgb_hi_ref,
             h_in_ref, z_ref, wtop_lo_ref, wtop_hi_ref, wbot_ref, cb_ref,
             u_ref, st2_ref):
    i = pl.program_id(0)
    inv = _inv_deg(deg_ref)
    a_lo, c_lo = _affine_from_stats(st_lo_ref, cgs_lo_ref, cgb_lo_ref)
    a_hi, c_hi = _affine_from_stats(st_hi_ref, cgs_hi_ref, cgb_hi_ref)
    m_lo = (agg_lo_ref[0] * inv) * a_lo + c_lo
    m_hi = (agg_hi_ref[0] * inv) * a_hi + c_hi
    cc = _mm(m_lo, wtop_lo_ref[...])
    cc += _mm(m_hi, wtop_hi_ref[...])
    cc += _mm(h_in_ref[...], wbot_ref[...])
    cc += cb_ref[...]
    mask = z_ref[...] > 0.5
    u = _blend(mask, cc[:, :H], cc[:, H:])
    u_ref[...] = u

    @pl.when(i == 0)
    def _():
        st2_ref[...] = jnp.zeros_like(st2_ref)

    st2_ref[0:1, :] += jnp.sum(u, axis=0, keepdims=True)
    st2_ref[1:2, :] += jnp.sum(u * u, axis=0, keepdims=True)


def _tc_p2(agg, deg2, st_lo, st_hi, cgs_lo, cgb_lo, cgs_hi, cgb_hi,
           h_in, z2d, wtop_lo, wtop_hi, wbot, cb):
    return pl.pallas_call(
        _p2_body,
        grid=(GRID,),
        in_specs=[
            pl.BlockSpec((1, BLK, HH), lambda i: (0, i, 0)),
            pl.BlockSpec((1, BLK, HH), lambda i: (1, i, 0)),
            pl.BlockSpec((BLK, 2), lambda i: (i, 0)),
            pl.BlockSpec((8, HH), lambda i: (0, 0)),
            pl.BlockSpec((8, HH), lambda i: (0, 0)),
            pl.BlockSpec((1, HH), lambda i: (0, 0)),
            pl.BlockSpec((1, HH), lambda i: (0, 0)),
            pl.BlockSpec((1, HH), lambda i: (0, 0)),
            pl.BlockSpec((1, HH), lambda i: (0, 0)),
            pl.BlockSpec((BLK, H), lambda i: (i, 0)),
            pl.BlockSpec((BLK, 1), lambda i: (i, 0)),
            pl.BlockSpec((HH, 2 * H), lambda i: (0, 0)),
            pl.BlockSpec((HH, 2 * H), lambda i: (0, 0)),
            pl.BlockSpec((H, 2 * H), lambda i: (0, 0)),
            pl.BlockSpec((1, 2 * H), lambda i: (0, 0)),
        ],
        out_specs=[
            pl.BlockSpec((BLK, H), lambda i: (i, 0)),
            pl.BlockSpec((8, H), lambda i: (0, 0)),
        ],
        out_shape=[
            jax.ShapeDtypeStruct((N, H), jnp.float32),
            jax.ShapeDtypeStruct((8, H), jnp.float32),
        ],
        compiler_params=pltpu.CompilerParams(
            dimension_semantics=("arbitrary",)),
    )(agg, agg, deg2, st_lo, st_hi, cgs_lo, cgb_lo, cgs_hi, cgb_hi,
      h_in, z2d, wtop_lo, wtop_hi, wbot, cb)


def _final_body(u_ref, stats_ref, gs_ref, gb_ref, o_ref):
    a, c = _affine_from_stats(stats_ref, gs_ref, gb_ref)
    o_ref[...] = u_ref[...] * a + c


def _tc_final(u, stats, gs, gb):
    return pl.pallas_call(
        _final_body,
        grid=(GRID,),
        in_specs=[
            pl.BlockSpec((BLK, H), lambda i: (i, 0)),
            pl.BlockSpec((8, H), lambda i: (0, 0)),
            pl.BlockSpec((1, H), lambda i: (0, 0)),
            pl.BlockSpec((1, H), lambda i: (0, 0)),
        ],
        out_specs=pl.BlockSpec((BLK, H), lambda i: (i, 0)),
        out_shape=jax.ShapeDtypeStruct((N, H), jnp.float32),
    )(u, stats, gs, gb)


def _sc_body(h2, colr, rowr, ewr, zrows, zdeg, out, degout,
             colbuf, rowbuf, ewbuf, idxbuf, gbuf, acc, dacc, sem,
             *, with_deg):
    c = lax.axis_index("c")
    s = lax.axis_index("s")

    @pl.when(s < 10)
    def _():
        pltpu.sync_copy(zrows, acc.at[pl.ds(s * 1000, 1000)])
    if with_deg:
        @pl.when(s == 10)
        def _():
            pltpu.sync_copy(zdeg, dacc)
    plsc.subcore_barrier()

    @pl.loop(0, NCHUNK)
    def _(i):
        base = s * EPT + (NCHUNK - 1 - i) * CHUNK
        pltpu.sync_copy(colr.at[pl.ds(base, CHUNK)], colbuf)
        pltpu.sync_copy(rowr.at[pl.ds(base, CHUNK)], rowbuf)
        pltpu.sync_copy(ewr.at[pl.ds(base, CHUNK)], ewbuf)
        for g in range(CHUNK // 16):
            idxbuf[pl.ds(g * 16, 16)] = colbuf[pl.ds(g * 16, 16)] * 2 + c
        pltpu.async_copy(h2.at[idxbuf], gbuf, sem).wait()

        @pl.loop(0, CHUNK // 16)
        def _(g):
            wv = ewbuf[pl.ds(g * 16, 16)]
            for lane in range(16):
                w = wv[lane]
                e = g * 16 + lane
                for j in range(HH // 16):
                    sl = pl.ds(j * 16, 16)
                    gbuf[e, sl] = gbuf[e, sl] * w
        pltpu.sync_copy(gbuf, acc.at[rowbuf], add=True)
        if with_deg:
            @pl.when((i % 2) == c)
            def _():
                pltpu.sync_copy(ewbuf, dacc.at[rowbuf], add=True)

    plsc.subcore_barrier()

    @pl.when(s < 10)
    def _():
        pltpu.sync_copy(acc.at[pl.ds(s * 1000, 1000)],
                        out.at[c, pl.ds(s * 1000, 1000)])

    if with_deg:
        @pl.when(s == 10)
        def _():
            pltpu.sync_copy(dacc, degout.at[c, 0])


def _sc_agg(h2, col, row, ew, zrows, zdeg, with_deg):
    out_type = [
        jax.ShapeDtypeStruct((NC, N, HH), jnp.float32),
        jax.ShapeDtypeStruct((NC, 1, N), jnp.float32),
    ]
    scratch = [
        pltpu.VMEM((CHUNK,), jnp.int32),
        pltpu.VMEM((CHUNK,), jnp.int32),
        pltpu.VMEM((CHUNK,), jnp.float32),
        pltpu.VMEM((CHUNK,), jnp.int32),
        pltpu.VMEM((CHUNK, HH), jnp.float32),
        pltpu.VMEM_SHARED((N, HH), jnp.float32),
        pltpu.VMEM_SHARED((N,), jnp.float32),
        pltpu.SemaphoreType.DMA,
    ]
    k = pl.kernel(
        functools.partial(_sc_body, with_deg=with_deg),
        out_type=out_type,
        mesh=_mesh(),
        scratch_types=scratch,
    )
    return k(h2, col, row, ew, zrows, zdeg)


def kernel(x, edge_index, edge_weight, z, emb_table, emb_gn_scale,
           emb_gn_bias, trans_W, trans_b, comb_W, comb_b, conv_gn_scale,
           conv_gn_bias, gns_scale, gns_bias):
    f32 = jnp.float32
    x2d = x.reshape(N, 1).astype(jnp.int32)
    z2d = z.reshape(N, 1)
    col = edge_index[1].astype(jnp.int32)
    row = edge_index[0].astype(jnp.int32)
    ew = edge_weight.astype(f32)
    emb_pad = jnp.zeros((128, H), f32).at[:emb_table.shape[0]].set(emb_table)
    zrows = jnp.zeros((1000, HH), f32)
    zdeg = jnp.zeros((N,), f32)

    wt = [jnp.concatenate([trans_W[l, 0], trans_W[l, 1]], axis=1)
          for l in range(3)]
    bt = [jnp.concatenate([trans_b[l, 0], trans_b[l, 1]]).reshape(1, 2 * H)
          for l in range(3)]
    wtop_lo = [jnp.concatenate([comb_W[l, 0][:HH], comb_W[l, 1][:HH]],
                               axis=1) for l in range(3)]
    wtop_hi = [jnp.concatenate([comb_W[l, 0][HH:H], comb_W[l, 1][HH:H]],
                               axis=1) for l in range(3)]
    wbot = [jnp.concatenate([comb_W[l, 0][H:], comb_W[l, 1][H:]],
                            axis=1) for l in range(3)]
    cb = [jnp.concatenate([comb_b[l, 0], comb_b[l, 1]]).reshape(1, 2 * H)
          for l in range(3)]
    cgs_lo = [conv_gn_scale[l, :HH].reshape(1, HH) for l in range(3)]
    cgs_hi = [conv_gn_scale[l, HH:].reshape(1, HH) for l in range(3)]
    cgb_lo = [conv_gn_bias[l, :HH].reshape(1, HH) for l in range(3)]
    cgb_hi = [conv_gn_bias[l, HH:].reshape(1, HH) for l in range(3)]
    gns_s = [gns_scale[l].reshape(1, H) for l in range(3)]
    gns_b = [gns_bias[l].reshape(1, H) for l in range(3)]
    egs = emb_gn_scale.reshape(1, H)
    egb = emb_gn_bias.reshape(1, H)

    h_pre, stats0 = _tc_emb(x2d, emb_pad)
    h_in, hm = _tc_trans(h_pre, stats0, egs, egb, z2d, wt[0], bt[0],
                         apply_relu=False)

    deg2 = None
    for l in range(3):
        agg, degout = _sc_agg(hm.reshape(2 * N, HH), col, row, ew,
                              zrows, zdeg, with_deg=(l == 0))
        if l == 0:
            deg2 = degout[:, 0, :].T
        st_lo, st_hi = _tc_p1(agg, deg2)
        u, st2 = _tc_p2(agg, deg2, st_lo, st_hi, cgs_lo[l], cgb_lo[l],
                        cgs_hi[l], cgb_hi[l], h_in, z2d,
                        wtop_lo[l], wtop_hi[l], wbot[l], cb[l])
        if l < 2:
            h_in, hm = _tc_trans(u, st2, gns_s[l], gns_b[l], z2d,
                                 wt[l + 1], bt[l + 1], apply_relu=True)
        else:
            return _tc_final(u, st2, gns_s[l], gns_b[l])


# speedup vs baseline: 3.8226x; 1.2819x over previous
import functools

import jax
import jax.numpy as jnp
from jax import lax
from jax.experimental import pallas as pl
from jax.experimental.pallas import tpu as pltpu
from jax.experimental.pallas import tpu_sc as plsc

N = 10000
E = 160000
H = 256
HH = 128
ZR = 0.8

NC = 2
NS = 16
EPT = E // (NS)
CHUNK = 80
NCHUNK = EPT // CHUNK
SUPER = 5

BLK = 1000
GRID = N // BLK

@functools.cache
def _mesh():
    return plsc.VectorSubcoreMesh(core_axis_name="c", subcore_axis_name="s",
                                  num_cores=NC, num_subcores=NS)


def _split_bf16(x):
    bits = lax.bitcast_convert_type(x, jnp.uint32)
    hi = lax.bitcast_convert_type(bits & jnp.uint32(0xFFFF0000), jnp.float32)
    lo = (x - hi) * 256.0
    return hi.astype(jnp.bfloat16), lo.astype(jnp.bfloat16)


def _mm(h, w):
    h_hi, h_lo = _split_bf16(h)
    w_hi, w_lo = _split_bf16(w)
    out = jnp.dot(h_hi, w_hi, preferred_element_type=jnp.float32)
    out += jnp.dot(h_lo, w_hi, preferred_element_type=jnp.float32) * (1.0 / 256.0)
    out += jnp.dot(h_hi, w_lo, preferred_element_type=jnp.float32) * (1.0 / 256.0)
    return out


def _blend(mask, a0, a1):
    return (1.0 - ZR) * (a0 + a1) + (2.0 * ZR - 1.0) * jnp.where(mask, a1, a0)


def _affine_from_stats(stats_ref, scale_ref, bias_ref):
    mean = stats_ref[0:1, :] * (1.0 / N)
    ex2 = stats_ref[1:2, :] * (1.0 / N)
    var = ex2 - mean * mean
    a = scale_ref[...] / jnp.sqrt(var + 1e-5)
    c = bias_ref[...] - mean * a
    return a, c


def _emb_body(x_ref, emb_ref, h_ref, stats_ref):
    i = pl.program_id(0)
    lanes = lax.broadcasted_iota(jnp.int32, (1, 128), 1)
    oh = (x_ref[...] == lanes).astype(jnp.float32)
    h = _mm(oh, emb_ref[...])

    @pl.when(i == 0)
    def _():
        stats_ref[...] = jnp.zeros_like(stats_ref)

    h_ref[...] = h
    stats_ref[0:1, :] += jnp.sum(h, axis=0, keepdims=True)
    stats_ref[1:2, :] += jnp.sum(h * h, axis=0, keepdims=True)


def _tc_emb(x2d, emb_pad):
    return pl.pallas_call(
        _emb_body,
        grid=(GRID,),
        in_specs=[
            pl.BlockSpec((BLK, 1), lambda i: (i, 0)),
            pl.BlockSpec((128, H), lambda i: (0, 0)),
        ],
        out_specs=[
            pl.BlockSpec((BLK, H), lambda i: (i, 0)),
            pl.BlockSpec((8, H), lambda i: (0, 0)),
        ],
        out_shape=[
            jax.ShapeDtypeStruct((N, H), jnp.float32),
            jax.ShapeDtypeStruct((8, H), jnp.float32),
        ],
        compiler_params=pltpu.CompilerParams(
            dimension_semantics=("arbitrary",)),
    )(x2d, emb_pad)


def _trans_body(u_ref, stats_ref, gs_ref, gb_ref, z_ref, wt_ref, bt_ref,
                h_ref, hm_ref, *, apply_relu):
    a, c = _affine_from_stats(stats_ref, gs_ref, gb_ref)
    h = u_ref[...] * a + c
    if apply_relu:
        h = jnp.maximum(h, 0.0)
    h_ref[...] = h
    xx = _mm(h, wt_ref[...]) + bt_ref[...]
    xx = jnp.maximum(xx, 0.0)
    mask = z_ref[...] > 0.5
    hm_ref[...] = _blend(mask, xx[:, :H], xx[:, H:])


def _tc_trans(u, stats, gs, gb, z2d, wt_bf16, bt, apply_relu):
    return pl.pallas_call(
        functools.partial(_trans_body, apply_relu=apply_relu),
        grid=(GRID,),
        in_specs=[
            pl.BlockSpec((BLK, H), lambda i: (i, 0)),
            pl.BlockSpec((8, H), lambda i: (0, 0)),
            pl.BlockSpec((1, H), lambda i: (0, 0)),
            pl.BlockSpec((1, H), lambda i: (0, 0)),
            pl.BlockSpec((BLK, 1), lambda i: (i, 0)),
            pl.BlockSpec((H, 2 * H), lambda i: (0, 0)),
            pl.BlockSpec((1, 2 * H), lambda i: (0, 0)),
        ],
        out_specs=[
            pl.BlockSpec((BLK, H), lambda i: (i, 0)),
            pl.BlockSpec((BLK, H), lambda i: (i, 0)),
        ],
        out_shape=[
            jax.ShapeDtypeStruct((N, H), jnp.float32),
            jax.ShapeDtypeStruct((N, H), jnp.float32),
        ],
    )(u, stats, gs, gb, z2d, wt_bf16, bt)


def _inv_deg(deg_ref):
    d = deg_ref[..., 0:1] + deg_ref[..., 1:2]
    d = jnp.where(d < 0.5, d + 1.0, d)
    return 1.0 / d


def _p1_body(agg_lo_ref, agg_hi_ref, deg_ref, st_lo_ref, st_hi_ref):
    i = pl.program_id(0)
    inv = _inv_deg(deg_ref)

    @pl.when(i == 0)
    def _():
        st_lo_ref[...] = jnp.zeros_like(st_lo_ref)
        st_hi_ref[...] = jnp.zeros_like(st_hi_ref)

    for aref, sref in ((agg_lo_ref, st_lo_ref), (agg_hi_ref, st_hi_ref)):
        s = aref[0] * inv
        sref[0:1, :] += jnp.sum(s, axis=0, keepdims=True)
        sref[1:2, :] += jnp.sum(s * s, axis=0, keepdims=True)


def _tc_p1(agg, deg2):
    return pl.pallas_call(
        _p1_body,
        grid=(GRID,),
        in_specs=[
            pl.BlockSpec((1, BLK, HH), lambda i: (0, i, 0)),
            pl.BlockSpec((1, BLK, HH), lambda i: (1, i, 0)),
            pl.BlockSpec((BLK, 2), lambda i: (i, 0)),
        ],
        out_specs=[
            pl.BlockSpec((8, HH), lambda i: (0, 0)),
            pl.BlockSpec((8, HH), lambda i: (0, 0)),
        ],
        out_shape=[
            jax.ShapeDtypeStruct((8, HH), jnp.float32),
            jax.ShapeDtypeStruct((8, HH), jnp.float32),
        ],
        compiler_params=pltpu.CompilerParams(
            dimension_semantics=("arbitrary",)),
    )(agg, agg, deg2)


def _p2_body(agg_lo_ref, agg_hi_ref, deg_ref, st_lo_ref, st_hi_ref,
             cgs_lo_ref, cgb_lo_ref, cgs_hi_ref, cgb_hi_ref,
             h_in_ref, z_ref, wtop_lo_ref, wtop_hi_ref, wbot_ref, cb_ref,
             u_ref, st2_ref):
    i = pl.program_id(0)
    inv = _inv_deg(deg_ref)
    a_lo, c_lo = _affine_from_stats(st_lo_ref, cgs_lo_ref, cgb_lo_ref)
    a_hi, c_hi = _affine_from_stats(st_hi_ref, cgs_hi_ref, cgb_hi_ref)
    m_lo = (agg_lo_ref[0] * inv) * a_lo + c_lo
    m_hi = (agg_hi_ref[0] * inv) * a_hi + c_hi
    cc = _mm(m_lo, wtop_lo_ref[...])
    cc += _mm(m_hi, wtop_hi_ref[...])
    cc += _mm(h_in_ref[...], wbot_ref[...])
    cc += cb_ref[...]
    mask = z_ref[...] > 0.5
    u = _blend(mask, cc[:, :H], cc[:, H:])
    u_ref[...] = u

    @pl.when(i == 0)
    def _():
        st2_ref[...] = jnp.zeros_like(st2_ref)

    st2_ref[0:1, :] += jnp.sum(u, axis=0, keepdims=True)
    st2_ref[1:2, :] += jnp.sum(u * u, axis=0, keepdims=True)


def _tc_p2(agg, deg2, st_lo, st_hi, cgs_lo, cgb_lo, cgs_hi, cgb_hi,
           h_in, z2d, wtop_lo, wtop_hi, wbot, cb):
    return pl.pallas_call(
        _p2_body,
        grid=(GRID,),
        in_specs=[
            pl.BlockSpec((1, BLK, HH), lambda i: (0, i, 0)),
            pl.BlockSpec((1, BLK, HH), lambda i: (1, i, 0)),
            pl.BlockSpec((BLK, 2), lambda i: (i, 0)),
            pl.BlockSpec((8, HH), lambda i: (0, 0)),
            pl.BlockSpec((8, HH), lambda i: (0, 0)),
            pl.BlockSpec((1, HH), lambda i: (0, 0)),
            pl.BlockSpec((1, HH), lambda i: (0, 0)),
            pl.BlockSpec((1, HH), lambda i: (0, 0)),
            pl.BlockSpec((1, HH), lambda i: (0, 0)),
            pl.BlockSpec((BLK, H), lambda i: (i, 0)),
            pl.BlockSpec((BLK, 1), lambda i: (i, 0)),
            pl.BlockSpec((HH, 2 * H), lambda i: (0, 0)),
            pl.BlockSpec((HH, 2 * H), lambda i: (0, 0)),
            pl.BlockSpec((H, 2 * H), lambda i: (0, 0)),
            pl.BlockSpec((1, 2 * H), lambda i: (0, 0)),
        ],
        out_specs=[
            pl.BlockSpec((BLK, H), lambda i: (i, 0)),
            pl.BlockSpec((8, H), lambda i: (0, 0)),
        ],
        out_shape=[
            jax.ShapeDtypeStruct((N, H), jnp.float32),
            jax.ShapeDtypeStruct((8, H), jnp.float32),
        ],
        compiler_params=pltpu.CompilerParams(
            dimension_semantics=("arbitrary",)),
    )(agg, agg, deg2, st_lo, st_hi, cgs_lo, cgb_lo, cgs_hi, cgb_hi,
      h_in, z2d, wtop_lo, wtop_hi, wbot, cb)


def _final_body(u_ref, stats_ref, gs_ref, gb_ref, o_ref):
    a, c = _affine_from_stats(stats_ref, gs_ref, gb_ref)
    o_ref[...] = u_ref[...] * a + c


def _tc_final(u, stats, gs, gb):
    return pl.pallas_call(
        _final_body,
        grid=(GRID,),
        in_specs=[
            pl.BlockSpec((BLK, H), lambda i: (i, 0)),
            pl.BlockSpec((8, H), lambda i: (0, 0)),
            pl.BlockSpec((1, H), lambda i: (0, 0)),
            pl.BlockSpec((1, H), lambda i: (0, 0)),
        ],
        out_specs=pl.BlockSpec((BLK, H), lambda i: (i, 0)),
        out_shape=jax.ShapeDtypeStruct((N, H), jnp.float32),
    )(u, stats, gs, gb)


def _sc_body(h2, colr, rowr, ewr, zrows, zdeg, out, degout,
             colsb, rowsb, ewsb, rowbuf, ewbuf, idxbuf, gbuf, acc, dacc,
             sem, *, with_deg):
    c = lax.axis_index("c")
    s = lax.axis_index("s")

    @pl.when(s < 10)
    def _():
        pltpu.sync_copy(zrows, acc.at[pl.ds(s * 1000, 1000)])
    if with_deg:
        @pl.when(s == 10)
        def _():
            pltpu.sync_copy(zdeg, dacc)
    plsc.subcore_barrier()

    @pl.loop(0, NCHUNK // SUPER)
    def _(b):
        base = s * EPT + b * (SUPER * CHUNK)
        pltpu.sync_copy(colr.at[pl.ds(base, SUPER * CHUNK)], colsb)
        pltpu.sync_copy(rowr.at[pl.ds(base, SUPER * CHUNK)], rowsb)
        pltpu.sync_copy(ewr.at[pl.ds(base, SUPER * CHUNK)], ewsb)
        for k in range(SUPER):
            i = b * SUPER + k
            for g in range(CHUNK // 16):
                sl = pl.ds(g * 16, 16)
                ssl = pl.ds(k * CHUNK + g * 16, 16)
                idxbuf[sl] = colsb[ssl] * 2 + c
                rowbuf[sl] = rowsb[ssl]
                ewbuf[sl] = ewsb[ssl]
            pltpu.async_copy(h2.at[idxbuf], gbuf, sem).wait()

            @pl.loop(0, CHUNK // 16)
            def _(g):
                wv = ewbuf[pl.ds(g * 16, 16)]
                for lane in range(16):
                    w = wv[lane]
                    e = g * 16 + lane
                    for j in range(HH // 16):
                        sl = pl.ds(j * 16, 16)
                        gbuf[e, sl] = gbuf[e, sl] * w
            pltpu.sync_copy(gbuf, acc.at[rowbuf], add=True)
            if with_deg:
                @pl.when((i % 2) == c)
                def _():
                    pltpu.sync_copy(ewbuf, dacc.at[rowbuf], add=True)

    plsc.subcore_barrier()

    @pl.when(s < 10)
    def _():
        pltpu.sync_copy(acc.at[pl.ds(s * 1000, 1000)],
                        out.at[c, pl.ds(s * 1000, 1000)])

    if with_deg:
        @pl.when(s == 10)
        def _():
            pltpu.sync_copy(dacc, degout.at[c, 0])


def _sc_agg(h2, col, row, ew, zrows, zdeg, with_deg):
    out_type = [
        jax.ShapeDtypeStruct((NC, N, HH), jnp.float32),
        jax.ShapeDtypeStruct((NC, 1, N), jnp.float32),
    ]
    scratch = [
        pltpu.VMEM((SUPER * CHUNK,), jnp.int32),
        pltpu.VMEM((SUPER * CHUNK,), jnp.int32),
        pltpu.VMEM((SUPER * CHUNK,), jnp.float32),
        pltpu.VMEM((CHUNK,), jnp.int32),
        pltpu.VMEM((CHUNK,), jnp.float32),
        pltpu.VMEM((CHUNK,), jnp.int32),
        pltpu.VMEM((CHUNK, HH), jnp.float32),
        pltpu.VMEM_SHARED((N, HH), jnp.float32),
        pltpu.VMEM_SHARED((N,), jnp.float32),
        pltpu.SemaphoreType.DMA,
    ]
    k = pl.kernel(
        functools.partial(_sc_body, with_deg=with_deg),
        out_type=out_type,
        mesh=_mesh(),
        scratch_types=scratch,
    )
    return k(h2, col, row, ew, zrows, zdeg)


def kernel(x, edge_index, edge_weight, z, emb_table, emb_gn_scale,
           emb_gn_bias, trans_W, trans_b, comb_W, comb_b, conv_gn_scale,
           conv_gn_bias, gns_scale, gns_bias):
    f32 = jnp.float32
    x2d = x.reshape(N, 1).astype(jnp.int32)
    z2d = z.reshape(N, 1)
    col = edge_index[1].astype(jnp.int32)
    row = edge_index[0].astype(jnp.int32)
    ew = edge_weight.astype(f32)
    emb_pad = jnp.zeros((128, H), f32).at[:emb_table.shape[0]].set(emb_table)
    zrows = jnp.zeros((1000, HH), f32)
    zdeg = jnp.zeros((N,), f32)

    wt = [jnp.concatenate([trans_W[l, 0], trans_W[l, 1]], axis=1)
          for l in range(3)]
    bt = [jnp.concatenate([trans_b[l, 0], trans_b[l, 1]]).reshape(1, 2 * H)
          for l in range(3)]
    wtop_lo = [jnp.concatenate([comb_W[l, 0][:HH], comb_W[l, 1][:HH]],
                               axis=1) for l in range(3)]
    wtop_hi = [jnp.concatenate([comb_W[l, 0][HH:H], comb_W[l, 1][HH:H]],
                               axis=1) for l in range(3)]
    wbot = [jnp.concatenate([comb_W[l, 0][H:], comb_W[l, 1][H:]],
                            axis=1) for l in range(3)]
    cb = [jnp.concatenate([comb_b[l, 0], comb_b[l, 1]]).reshape(1, 2 * H)
          for l in range(3)]
    cgs_lo = [conv_gn_scale[l, :HH].reshape(1, HH) for l in range(3)]
    cgs_hi = [conv_gn_scale[l, HH:].reshape(1, HH) for l in range(3)]
    cgb_lo = [conv_gn_bias[l, :HH].reshape(1, HH) for l in range(3)]
    cgb_hi = [conv_gn_bias[l, HH:].reshape(1, HH) for l in range(3)]
    gns_s = [gns_scale[l].reshape(1, H) for l in range(3)]
    gns_b = [gns_bias[l].reshape(1, H) for l in range(3)]
    egs = emb_gn_scale.reshape(1, H)
    egb = emb_gn_bias.reshape(1, H)

    h_pre, stats0 = _tc_emb(x2d, emb_pad)
    h_in, hm = _tc_trans(h_pre, stats0, egs, egb, z2d, wt[0], bt[0],
                         apply_relu=False)

    deg2 = None
    for l in range(3):
        agg, degout = _sc_agg(hm.reshape(2 * N, HH), col, row, ew,
                              zrows, zdeg, with_deg=(l == 0))
        if l == 0:
            deg2 = degout[:, 0, :].T
        st_lo, st_hi = _tc_p1(agg, deg2)
        u, st2 = _tc_p2(agg, deg2, st_lo, st_hi, cgs_lo[l], cgb_lo[l],
                        cgs_hi[l], cgb_hi[l], h_in, z2d,
                        wtop_lo[l], wtop_hi[l], wbot[l], cb[l])
        if l < 2:
            h_in, hm = _tc_trans(u, st2, gns_s[l], gns_b[l], z2d,
                                 wt[l + 1], bt[l + 1], apply_relu=True)
        else:
            return _tc_final(u, st2, gns_s[l], gns_b[l])
